# baseline (speedup 1.0000x reference)
import functools

import jax
import jax.numpy as jnp
from jax import lax
from jax.experimental import pallas as pl
from jax.experimental.pallas import tpu as pltpu
from jax.experimental.pallas import tpu_sc as plsc

DIAM = 512
EDIM = 64


def _renorm_body(t_ref, o_ref):
    t = t_ref[...]
    norms = jnp.max(jnp.abs(t), axis=1, keepdims=True)
    scale = jnp.where(norms > 1.0, 1.0 / (norms + 1e-7), 1.0)
    o_ref[...] = t * scale


def _renorm(table):
    return pl.pallas_call(
        _renorm_body,
        out_shape=jax.ShapeDtypeStruct(table.shape, table.dtype),
    )(table)


def _sc_gather(idx2d, table):
    n_rows, CH = idx2d.shape
    NW = 32
    n_ch = n_rows // NW
    NB = 10
    K = 5
    n_grp = n_ch // NB
    assert n_ch * NW == n_rows and n_grp * NB == n_ch
    B = n_rows * CH

    mesh = plsc.VectorSubcoreMesh(core_axis_name="c", subcore_axis_name="s")

    @functools.partial(
        pl.kernel,
        mesh=mesh,
        compiler_params=pltpu.CompilerParams(use_tc_tiling_on_sc=False),
        out_type=jax.ShapeDtypeStruct((B, EDIM), jnp.float32),
        scratch_types=[
            pltpu.VMEM((n_ch, CH), jnp.int32),
            pltpu.VMEM((NB, CH, EDIM), jnp.float32),
            pltpu.SemaphoreType.DMA,
            pltpu.SemaphoreType.DMA((NB,)),
            pltpu.SemaphoreType.DMA((NB,)),
        ],
    )
    def k(idx_hbm, tbl_hbm, out_hbm, idx_v, rows_v, sem_i, sem_g, sem_s):
        wid = lax.axis_index("s") * 2 + lax.axis_index("c")
        row0 = wid * n_ch

        copy_i = pltpu.make_async_copy(
            idx_hbm.at[pl.ds(row0, n_ch)], idx_v, sem_i)
        copy_i.start()
        copy_i.wait()

        def clamp_row(j, c):
            for i in range(CH // 16):
                sl = pl.ds(i * 16, 16)
                idx_v[j, sl] = jnp.minimum(idx_v[j, sl], DIAM)
            return c

        lax.fori_loop(0, n_ch, clamp_row, 0)

        def fire_gather(j, b):
            pltpu.make_async_copy(
                tbl_hbm.at[idx_v.at[j]], rows_v.at[b], sem_g.at[b]).start()

        def wait_gather(b):
            pltpu.make_async_copy(
                tbl_hbm.at[idx_v.at[0]], rows_v.at[b], sem_g.at[b]).wait()

        def fire_store(j, b):
            pltpu.make_async_copy(
                rows_v.at[b], out_hbm.at[pl.ds((row0 + j) * CH, CH)],
                sem_s.at[b]).start()

        def wait_store(b):
            pltpu.make_async_copy(
                rows_v.at[b], out_hbm.at[pl.ds(0, CH)], sem_s.at[b]).wait()

        for b in range(K):
            fire_gather(b, b)

        def group(g, c):
            i0 = g * NB
            for b in range(NB):
                i = i0 + b
                wait_gather(b)
                fire_store(i, b)
                bb = (b + K) % NB

                @pl.when(i + K < n_ch)
                def _():
                    @pl.when(i + K >= NB)
                    def _():
                        wait_store(bb)

                    fire_gather(i + K, bb)
            return c

        lax.fori_loop(0, n_grp, group, 0)

        for b in range(NB):
            wait_store(b)

    return k(idx2d, table)


def kernel(x, table):
    renormed = _renorm(table)
    out = _sc_gather(x.reshape(-1, 128), renormed)
    return out.reshape(x.shape + (EDIM,))


# speedup vs baseline: 3.5804x; 3.5804x over previous
import functools

import jax
import jax.numpy as jnp
from jax import lax
from jax.experimental import pallas as pl
from jax.experimental.pallas import tpu as pltpu
from jax.experimental.pallas import tpu_sc as plsc

DIAM = 512
EDIM = 64


def _renorm_body(t_ref, o_ref):
    t = t_ref[...]
    norms = jnp.max(jnp.abs(t), axis=1, keepdims=True)
    scale = jnp.where(norms > 1.0, 1.0 / (norms + 1e-7), 1.0)
    o_ref[...] = t * scale


def _renorm(table):
    return pl.pallas_call(
        _renorm_body,
        out_shape=jax.ShapeDtypeStruct(table.shape, table.dtype),
    )(table)


def _sc_gather(idx2d, table):
    n_rows, CH = idx2d.shape
    V = table.shape[0]
    NW = 32
    n_ch = n_rows // NW
    NB = 4
    RU = 4
    n_grp = n_ch // NB
    assert n_ch * NW == n_rows and n_grp * NB == n_ch
    B = n_rows * CH

    mesh = plsc.VectorSubcoreMesh(core_axis_name="c", subcore_axis_name="s")

    @functools.partial(
        pl.kernel,
        mesh=mesh,
        compiler_params=pltpu.CompilerParams(use_tc_tiling_on_sc=False),
        out_type=jax.ShapeDtypeStruct((B, EDIM), jnp.float32),
        scratch_types=[
            pltpu.VMEM((V, EDIM), jnp.float32),
            pltpu.VMEM((n_ch, CH), jnp.int32),
            pltpu.VMEM((NB, CH, EDIM), jnp.float32),
            pltpu.SemaphoreType.DMA,
            pltpu.SemaphoreType.DMA,
            pltpu.SemaphoreType.DMA((NB,)),
        ],
    )
    def k(idx_hbm, tbl_hbm, out_hbm, tbl_v, idx_v, rows_v, sem_t, sem_i,
          sem_s):
        wid = lax.axis_index("s") * 2 + lax.axis_index("c")
        row0 = wid * n_ch

        copy_t = pltpu.make_async_copy(tbl_hbm, tbl_v, sem_t)
        copy_t.start()
        copy_i = pltpu.make_async_copy(
            idx_hbm.at[pl.ds(row0, n_ch)], idx_v, sem_i)
        copy_i.start()
        copy_t.wait()
        copy_i.wait()

        def wait_store(b):
            pltpu.make_async_copy(
                rows_v.at[b], out_hbm.at[pl.ds(0, CH)], sem_s.at[b]).wait()

        def gather_chunk(j, b):
            def rows(r0, c):
                iv = jnp.minimum(idx_v[j, pl.ds(r0 * 16, 16)], DIAM)
                for u in range(16):
                    r = r0 * 16 + u
                    i = iv[u]
                    for kk in range(EDIM // 16):
                        sl = pl.ds(kk * 16, 16)
                        rows_v[b, r, sl] = tbl_v[i, sl]
                return c

            lax.fori_loop(0, CH // 16, rows, 0)

        def group(g, c):
            j0 = g * NB
            for b in range(NB):
                j = j0 + b

                @pl.when(j >= NB)
                def _():
                    wait_store(b)

                gather_chunk(j, b)
                pltpu.make_async_copy(
                    rows_v.at[b], out_hbm.at[pl.ds((row0 + j) * CH, CH)],
                    sem_s.at[b]).start()
            return c

        lax.fori_loop(0, n_grp, group, 0)

        for b in range(NB):
            wait_store(b)

    return k(idx2d, table)


def kernel(x, table):
    renormed = _renorm(table)
    out = _sc_gather(x.reshape(-1, 128), renormed)
    return out.reshape(x.shape + (EDIM,))


# speedup vs baseline: 5.4606x; 1.5251x over previous
import functools

import jax
import jax.numpy as jnp
from jax import lax
from jax.experimental import pallas as pl
from jax.experimental.pallas import tpu as pltpu
from jax.experimental.pallas import tpu_sc as plsc

DIAM = 512
EDIM = 64


def _renorm_body(t_ref, o_ref):
    t = t_ref[...]
    norms = jnp.max(jnp.abs(t), axis=1, keepdims=True)
    scale = jnp.where(norms > 1.0, 1.0 / (norms + 1e-7), 1.0)
    o_ref[...] = t * scale


def _renorm(table):
    return pl.pallas_call(
        _renorm_body,
        out_shape=jax.ShapeDtypeStruct(table.shape, table.dtype),
    )(table)


def _sc_gather(idx2d, table):
    n_rows, CH = idx2d.shape
    V = table.shape[0]
    NW = 32
    n_ch = n_rows // NW
    NB = 2
    RU = 4
    n_grp = n_ch // NB
    assert n_ch * NW == n_rows and n_grp * NB == n_ch
    B = n_rows * CH

    mesh = plsc.VectorSubcoreMesh(core_axis_name="c", subcore_axis_name="s")

    @functools.partial(
        pl.kernel,
        mesh=mesh,
        out_type=jax.ShapeDtypeStruct((B, EDIM), jnp.float32),
        scratch_types=[
            pltpu.VMEM((V, EDIM), jnp.float32),
            pltpu.VMEM((n_ch, CH), jnp.int32),
            pltpu.VMEM((NB, CH, EDIM), jnp.float32),
            pltpu.SemaphoreType.DMA,
            pltpu.SemaphoreType.DMA,
            pltpu.SemaphoreType.DMA((NB,)),
        ],
    )
    def k(idx_hbm, tbl_hbm, out_hbm, tbl_v, idx_v, rows_v, sem_t, sem_i,
          sem_s):
        wid = lax.axis_index("s") * 2 + lax.axis_index("c")
        row0 = wid * n_ch

        copy_t = pltpu.make_async_copy(tbl_hbm, tbl_v, sem_t)
        copy_t.start()
        copy_i = pltpu.make_async_copy(
            idx_hbm.at[pl.ds(row0, n_ch)], idx_v, sem_i)
        copy_i.start()
        copy_t.wait()
        copy_i.wait()

        def wait_store(b):
            pltpu.make_async_copy(
                rows_v.at[b], out_hbm.at[pl.ds(0, CH)], sem_s.at[b]).wait()

        def gather_chunk(j, b):
            def rows(r0, c):
                iv = jnp.minimum(idx_v[j, pl.ds(r0 * 16, 16)], DIAM)
                for u in range(16):
                    r = r0 * 16 + u
                    i = iv[u]
                    for kk in range(EDIM // 16):
                        sl = pl.ds(kk * 16, 16)
                        rows_v[b, r, sl] = tbl_v[i, sl]
                return c

            lax.fori_loop(0, CH // 16, rows, 0)

        def group(g, c):
            j0 = g * NB
            for b in range(NB):
                j = j0 + b

                @pl.when(j >= NB)
                def _():
                    wait_store(b)

                gather_chunk(j, b)
                pltpu.make_async_copy(
                    rows_v.at[b], out_hbm.at[pl.ds((row0 + j) * CH, CH)],
                    sem_s.at[b]).start()
            return c

        lax.fori_loop(0, n_grp, group, 0)

        for b in range(NB):
            wait_store(b)

    return k(idx2d, table)


def kernel(x, table):
    renormed = _renorm(table)
    out = _sc_gather(x.reshape(-1, 128), renormed)
    return out.reshape(x.shape + (EDIM,))


# speedup vs baseline: 7.8575x; 1.4389x over previous
import functools

import jax
import jax.numpy as jnp
from jax import lax
from jax.experimental import pallas as pl
from jax.experimental.pallas import tpu as pltpu
from jax.experimental.pallas import tpu_sc as plsc

DIAM = 512
EDIM = 64


def _renorm_body(t_ref, o_ref):
    t = t_ref[...]
    norms = jnp.max(jnp.abs(t), axis=1, keepdims=True)
    scale = jnp.where(norms > 1.0, 1.0 / (norms + 1e-7), 1.0)
    o_ref[...] = t * scale


def _renorm(table):
    return pl.pallas_call(
        _renorm_body,
        out_shape=jax.ShapeDtypeStruct(table.shape, table.dtype),
    )(table)


def _sc_gather(idx2d, table):
    n_rows, CH = idx2d.shape
    V = table.shape[0]
    NW = 32
    n_ch = n_rows // NW
    NB = 2
    RU = 4
    n_grp = n_ch // NB
    assert n_ch * NW == n_rows and n_grp * NB == n_ch
    B = n_rows * CH

    mesh = plsc.VectorSubcoreMesh(core_axis_name="c", subcore_axis_name="s")

    @functools.partial(
        pl.kernel,
        mesh=mesh,
        out_type=jax.ShapeDtypeStruct((B, EDIM), jnp.float32),
        scratch_types=[
            pltpu.VMEM((V, EDIM), jnp.float32),
            pltpu.VMEM((n_ch, CH), jnp.int32),
            pltpu.VMEM((NB, CH, EDIM), jnp.float32),
            pltpu.SemaphoreType.DMA,
            pltpu.SemaphoreType.DMA,
            pltpu.SemaphoreType.DMA((NB,)),
        ],
    )
    def k(idx_hbm, tbl_hbm, out_hbm, tbl_v, idx_v, rows_v, sem_t, sem_i,
          sem_s):
        wid = lax.axis_index("s") * 2 + lax.axis_index("c")
        row0 = wid * n_ch

        copy_t = pltpu.make_async_copy(tbl_hbm, tbl_v, sem_t)
        copy_t.start()
        copy_i = pltpu.make_async_copy(
            idx_hbm.at[pl.ds(row0, n_ch)], idx_v, sem_i)
        copy_i.start()
        copy_t.wait()
        copy_i.wait()

        def wait_store(b):
            pltpu.make_async_copy(
                rows_v.at[b], out_hbm.at[pl.ds(0, CH)], sem_s.at[b]).wait()

        def gather_chunk(j, b):
            @plsc.parallel_loop(0, CH // 16, unroll=2)
            def rows(r0):
                iv = jnp.minimum(idx_v[j, pl.ds(r0 * 16, 16)], DIAM)
                for u in range(16):
                    r = r0 * 16 + u
                    i = iv[u]
                    for kk in range(EDIM // 16):
                        sl = pl.ds(kk * 16, 16)
                        rows_v[b, r, sl] = tbl_v[i, sl]

        def group(g, c):
            j0 = g * NB
            for b in range(NB):
                j = j0 + b

                @pl.when(j >= NB)
                def _():
                    wait_store(b)

                gather_chunk(j, b)
                pltpu.make_async_copy(
                    rows_v.at[b], out_hbm.at[pl.ds((row0 + j) * CH, CH)],
                    sem_s.at[b]).start()
            return c

        lax.fori_loop(0, n_grp, group, 0)

        for b in range(NB):
            wait_store(b)

    return k(idx2d, table)


def kernel(x, table):
    renormed = _renorm(table)
    out = _sc_gather(x.reshape(-1, 128), renormed)
    return out.reshape(x.shape + (EDIM,))


# speedup vs baseline: 8.4070x; 1.0699x over previous
import functools

import jax
import jax.numpy as jnp
from jax import lax
from jax.experimental import pallas as pl
from jax.experimental.pallas import tpu as pltpu
from jax.experimental.pallas import tpu_sc as plsc

DIAM = 512
EDIM = 64
TSTRIDE = 65


def _renorm_body(t_ref, o_ref):
    t = t_ref[...]
    norms = jnp.max(jnp.abs(t), axis=1, keepdims=True)
    scale = jnp.where(norms > 1.0, 1.0 / (norms + 1e-7), 1.0)
    o_ref[...] = t * scale


def _renorm(table):
    return pl.pallas_call(
        _renorm_body,
        out_shape=jax.ShapeDtypeStruct(table.shape, table.dtype),
    )(table)


def _sc_gather_t(x, table):
    NB_, T = x.shape
    V = table.shape[0]
    NW = 32
    BPW = NB_ // NW
    NB = 2
    n_tg = T // NB
    assert BPW * NW == NB_ and n_tg * NB == T

    mesh = plsc.VectorSubcoreMesh(core_axis_name="c", subcore_axis_name="s")

    @functools.partial(
        pl.kernel,
        mesh=mesh,
        compiler_params=pltpu.CompilerParams(needs_layout_passes=False),
        out_type=jax.ShapeDtypeStruct((T, EDIM, NB_), jnp.float32),
        scratch_types=[
            pltpu.VMEM((64, EDIM), jnp.float32),
            pltpu.VMEM((V * TSTRIDE,), jnp.float32),
            pltpu.VMEM((BPW, T), jnp.int32),
            pltpu.VMEM((T, BPW), jnp.int32),
            pltpu.VMEM((NB, EDIM, BPW), jnp.float32),
            pltpu.SemaphoreType.DMA,
            pltpu.SemaphoreType.DMA,
            pltpu.SemaphoreType.DMA((NB,)),
        ],
    )
    def k(x_hbm, tbl_hbm, out_hbm, stg_v, tbl_v, idx_v, idxt_v, tiles_v,
          sem_t, sem_i, sem_s):
        wid = lax.axis_index("s") * 2 + lax.axis_index("c")
        b0 = wid * BPW
        lanes = lax.broadcasted_iota(jnp.int32, (16,), 0)

        copy_i = pltpu.make_async_copy(
            x_hbm.at[pl.ds(b0, BPW)], idx_v, sem_i)
        copy_i.start()

        for s in range(9):
            r0, nr = (s * 64, 64) if s < 8 else (512, 1)
            copy_t = pltpu.make_async_copy(
                tbl_hbm.at[pl.ds(r0, nr)], stg_v.at[pl.ds(0, nr)], sem_t)
            copy_t.start()
            copy_t.wait()

            def repack(r, c):
                base = (r0 + r) * TSTRIDE
                for kk in range(EDIM // 16):
                    v = stg_v[r, pl.ds(kk * 16, 16)]
                    plsc.store_scatter(
                        tbl_v, [base + kk * 16 + lanes], v)
                return c

            lax.fori_loop(0, nr, repack, 0)

        copy_i.wait()

        def tr_row(t, c):
            tvec = jnp.full((16,), t, dtype=jnp.int32)
            for g in range(BPW // 16):
                v = plsc.load_gather(idx_v, [g * 16 + lanes, tvec])
                idxt_v[t, pl.ds(g * 16, 16)] = v
            return c

        lax.fori_loop(0, T, tr_row, 0)

        def wait_store(nb):
            pltpu.make_async_copy(
                tiles_v.at[nb], out_hbm.at[0, :, pl.ds(0, BPW)],
                sem_s.at[nb]).wait()

        def fill_tile(t, nb):
            @plsc.parallel_loop(0, BPW // 16, unroll=2)
            def grp(g):
                iv = jnp.minimum(idxt_v[t, pl.ds(g * 16, 16)], DIAM)
                a = iv * TSTRIDE
                for d in range(EDIM):
                    v = plsc.load_gather(tbl_v, [a + d])
                    tiles_v[nb, d, pl.ds(g * 16, 16)] = v

        def tgroup(tg, c):
            t0 = tg * NB
            for nb in range(NB):
                t = t0 + nb

                @pl.when(t >= NB)
                def _():
                    wait_store(nb)

                fill_tile(t, nb)
                pltpu.make_async_copy(
                    tiles_v.at[nb], out_hbm.at[t, :, pl.ds(b0, BPW)],
                    sem_s.at[nb]).start()
            return c

        lax.fori_loop(0, n_tg, tgroup, 0)

        for nb in range(NB):
            wait_store(nb)

    return k(x, table)


def kernel(x, table):
    renormed = _renorm(table)
    out_t = _sc_gather_t(x, renormed)
    return jnp.transpose(out_t, (2, 0, 1))


# speedup vs baseline: 14.1808x; 1.6868x over previous
import functools

import jax
import jax.numpy as jnp
from jax import lax
from jax.experimental import pallas as pl
from jax.experimental.pallas import tpu as pltpu
from jax.experimental.pallas import tpu_sc as plsc

DIAM = 512
EDIM = 64
TSTRIDE = 65


def _renorm_body(t_ref, o_ref):
    t = t_ref[...]
    norms = jnp.max(jnp.abs(t), axis=1, keepdims=True)
    scale = jnp.where(norms > 1.0, 1.0 / (norms + 1e-7), 1.0)
    o_ref[...] = t * scale


def _renorm(table):
    return pl.pallas_call(
        _renorm_body,
        out_shape=jax.ShapeDtypeStruct(table.shape, table.dtype),
    )(table)


def _sc_gather_t(x, table):
    NB_, T = x.shape
    V = table.shape[0]
    NW = 32
    BPW = NB_ // NW
    NB = 2
    n_tg = T // NB
    assert BPW * NW == NB_ and n_tg * NB == T

    mesh = plsc.VectorSubcoreMesh(core_axis_name="c", subcore_axis_name="s")

    @functools.partial(
        pl.kernel,
        mesh=mesh,
        compiler_params=pltpu.CompilerParams(needs_layout_passes=False),
        out_type=jax.ShapeDtypeStruct((T, EDIM, NB_), jnp.float32),
        scratch_types=[
            pltpu.VMEM((64, EDIM), jnp.float32),
            pltpu.VMEM((V * TSTRIDE,), jnp.float32),
            pltpu.VMEM((BPW, T), jnp.int32),
            pltpu.VMEM((T, BPW), jnp.int32),
            pltpu.VMEM((NB, EDIM, BPW), jnp.float32),
            pltpu.SemaphoreType.DMA,
            pltpu.SemaphoreType.DMA,
            pltpu.SemaphoreType.DMA((NB,)),
        ],
    )
    def k(x_hbm, tbl_hbm, out_hbm, stg_v, tbl_v, idx_v, idxt_v, tiles_v,
          sem_t, sem_i, sem_s):
        wid = lax.axis_index("s") * 2 + lax.axis_index("c")
        b0 = wid * BPW
        lanes = lax.broadcasted_iota(jnp.int32, (16,), 0)


        for s in range(9):
            r0, nr = (s * 64, 64) if s < 8 else (512, 1)
            copy_t = pltpu.make_async_copy(
                tbl_hbm.at[pl.ds(r0, nr)], stg_v.at[pl.ds(0, nr)], sem_t)
            copy_t.start()
            copy_t.wait()

            def repack(r, c):
                base = (r0 + r) * TSTRIDE
                for kk in range(EDIM // 16):
                    v = stg_v[r, pl.ds(kk * 16, 16)]
                    plsc.store_scatter(
                        tbl_v, [base + kk * 16 + lanes], v)
                return c

            lax.fori_loop(0, nr, repack, 0)

        copy_i = pltpu.make_async_copy(x_hbm.at[pl.ds(b0, BPW)], idx_v, sem_i)
        copy_i.start()
        copy_i.wait()

        def tr_row(t, c):
            tvec = jnp.full((16,), t, dtype=jnp.int32)
            for g in range(BPW // 16):
                v = plsc.load_gather(idx_v, [g * 16 + lanes, tvec])
                idxt_v[t, pl.ds(g * 16, 16)] = v
            return c

        lax.fori_loop(0, T, tr_row, 0)

        def wait_store(nb):
            pltpu.make_async_copy(
                tiles_v.at[nb], out_hbm.at[0, :, pl.ds(0, BPW)],
                sem_s.at[nb]).wait()

        def fill_tile(t, nb):
            @plsc.parallel_loop(0, BPW // 16, unroll=4)
            def grp(g):
                iv = jnp.minimum(idxt_v[t, pl.ds(g * 16, 16)], DIAM)
                a = iv * TSTRIDE
                for d in range(EDIM):
                    v = plsc.load_gather(tbl_v, [a + d])
                    tiles_v[nb, d, pl.ds(g * 16, 16)] = v

        def tgroup(tg, c):
            t0 = tg * NB
            for nb in range(NB):
                t = t0 + nb

                @pl.when(t >= NB)
                def _():
                    wait_store(nb)

                fill_tile(t, nb)
                pltpu.make_async_copy(
                    tiles_v.at[nb], out_hbm.at[t, :, pl.ds(b0, BPW)],
                    sem_s.at[nb]).start()
            return c

        lax.fori_loop(0, n_tg, tgroup, 0)

        for nb in range(NB):
            wait_store(nb)

    return k(x, table)


def kernel(x, table):
    renormed = _renorm(table)
    out_t = _sc_gather_t(x, renormed)
    return jnp.transpose(out_t, (2, 0, 1))
